# baseline (speedup 1.0000x reference)
import functools

import jax
import jax.numpy as jnp
from jax import lax
from jax.experimental import pallas as pl
from jax.experimental.pallas import tpu as pltpu
from jax.experimental.pallas import tpu_sc as plsc

_NC = 2
_NS = 16
_NW = _NC * _NS
_L = 16
_LANES = 128

_CHUNK = 4096
_SC_UNITS = 16
_TC_BLOCK = 512


def _take(tab, idx):
    return tab.at[idx].get(mode="promise_in_bounds")


@functools.lru_cache(maxsize=None)
def _build_sc_kernel(n: int, k: int, n_sc: int):
    assert k == 8, "kernel is specialized to K == 8 mixture components"
    per_w = n_sc // _NW
    assert per_w * _NW == n_sc
    chunk = min(_CHUNK, per_w)
    n_ch = per_w // chunk
    assert n_ch * chunk == per_w
    assert chunk % _LANES == 0 and n % _LANES == 0
    tile = k * _LANES

    mesh = plsc.VectorSubcoreMesh(
        core_axis_name="c", subcore_axis_name="s", num_cores=_NC, num_subcores=_NS
    )

    @functools.partial(
        pl.kernel,
        mesh=mesh,
        compiler_params=pltpu.CompilerParams(needs_layout_passes=False),
        out_type=jax.ShapeDtypeStruct((n_sc,), jnp.float32),
        scratch_types=[
            pltpu.VMEM((chunk * k,), jnp.float32),
            pltpu.VMEM((chunk * k,), jnp.float32),
            pltpu.VMEM((chunk,), jnp.int32),
            pltpu.VMEM((chunk,), jnp.int32),
            pltpu.VMEM((chunk,), jnp.float32),
            pltpu.VMEM((chunk,), jnp.float32),
            pltpu.VMEM((2 * k,), jnp.float32),
            pltpu.SemaphoreType.DMA,
            pltpu.SemaphoreType.DMA,
            pltpu.SemaphoreType.DMA,
            pltpu.SemaphoreType.DMA,
        ],
    )
    def sc_kernel(eps_hbm, ms_hbm, tab_hbm, out_hbm,
                  eb0, eb1, mb0, mb1, ob0, ob1, tab_v, ls0, ls1, ss0, ss1):
        eb = (eb0, eb1)
        mb = (mb0, mb1)
        ob = (ob0, ob1)
        lsem = (ls0, ls1)
        ssem = (ss0, ss1)

        wid = lax.axis_index("s") * _NC + lax.axis_index("c")
        base = wid * per_w

        pltpu.sync_copy(tab_hbm, tab_v)
        tab = tab_v[...]

        iota = lax.iota(jnp.int32, _L)

        def start_loads(c, b):
            off = base + c * chunk
            d1 = pltpu.async_copy(
                eps_hbm.at[pl.ds(off * k, chunk * k)], eb[b], lsem[b]
            )
            d2 = pltpu.async_copy(ms_hbm.at[pl.ds(off, chunk)], mb[b], lsem[b])
            return (d1, d2)

        def compute(b):
            @plsc.parallel_loop(0, chunk, _L, unroll=8)
            def p(j):
                sl = pl.ds(j, _L)
                m = mb[b][sl]
                lo = _take(tab, m)
                sc = _take(tab, m + k)
                s = (j // _LANES) * tile + (j % _LANES)
                lidx = lax.shift_left(m, 7) + (s + iota)
                g = plsc.load_gather(eb[b], [lidx])
                ob[b][sl] = lo + sc * g

        def start_store(c, b):
            off = base + c * chunk
            return pltpu.async_copy(ob[b], out_hbm.at[pl.ds(off, chunk)], ssem[b])

        load_d = [None, None]
        store_d = [None, None]
        load_d[0] = start_loads(0, 0)
        if n_ch > 1:
            load_d[1] = start_loads(1, 1)
        for c in range(n_ch):
            b = c & 1
            for d in load_d[b]:
                d.wait()
            if store_d[b] is not None:
                store_d[b].wait()
                store_d[b] = None
            compute(b)
            store_d[b] = start_store(c, b)
            if c + 2 < n_ch:
                load_d[b] = start_loads(c + 2, b)
        for b in range(2):
            if store_d[b] is not None:
                store_d[b].wait()

    return sc_kernel


@functools.lru_cache(maxsize=None)
def _build_tc_kernel(n: int, k: int, n_sc: int):
    t_all = n // _LANES
    t_sc = n_sc // _LANES
    t_tc = t_all - t_sc
    rb = min(_TC_BLOCK, t_tc)
    assert t_tc % rb == 0 and t_sc % rb == 0
    grid = (t_tc // rb,)
    off_b = t_sc // rb

    def tc_body(eps_ref, ms_ref, loc_ref, scale_ref, out_ref):
        m = ms_ref[...]
        acc = loc_ref[0] + scale_ref[0] * eps_ref[:, 0, :]
        for mm in range(1, k):
            acc = jnp.where(
                m == mm, loc_ref[mm] + scale_ref[mm] * eps_ref[:, mm, :], acc
            )
        out_ref[...] = acc

    return pl.pallas_call(
        tc_body,
        grid=grid,
        in_specs=[
            pl.BlockSpec((rb, k, _LANES), lambda b: (off_b + b, 0, 0)),
            pl.BlockSpec((rb, _LANES), lambda b: (off_b + b, 0)),
            pl.BlockSpec(memory_space=pltpu.SMEM),
            pl.BlockSpec(memory_space=pltpu.SMEM),
        ],
        out_specs=pl.BlockSpec((rb, _LANES), lambda b: (b, 0)),
        out_shape=jax.ShapeDtypeStruct((t_tc, _LANES), jnp.float32),
    )


def kernel(eps, ms, location, scale):
    n, k = eps.shape
    n_sc = (n // 32) * _SC_UNITS
    assert n_sc % (_NW * _LANES) == 0
    eps3 = eps.reshape(n // _LANES, _LANES, k).transpose(0, 2, 1)
    ms_i = ms.astype(jnp.int32)
    loc_f = location.astype(jnp.float32)
    scale_f = scale.astype(jnp.float32)
    tab = jnp.concatenate([loc_f, scale_f])

    sc_out = _build_sc_kernel(n, k, n_sc)(
        eps3.reshape(n * k), ms_i, tab
    )
    tc_out = _build_tc_kernel(n, k, n_sc)(
        eps3, ms_i.reshape(n // _LANES, _LANES), loc_f, scale_f
    )
    return jnp.concatenate([sc_out, tc_out.reshape(n - n_sc)])


# speedup vs baseline: 1.1212x; 1.1212x over previous
import functools

import jax
import jax.numpy as jnp
from jax import lax
from jax.experimental import pallas as pl
from jax.experimental.pallas import tpu as pltpu
from jax.experimental.pallas import tpu_sc as plsc

_NC = 2
_NS = 16
_NW = _NC * _NS
_L = 16
_LANES = 128

_CHUNK = 4096
_SC_UNITS = 24
_TC_BLOCK = 512


def _take(tab, idx):
    return tab.at[idx].get(mode="promise_in_bounds")


@functools.lru_cache(maxsize=None)
def _build_sc_kernel(n: int, k: int, n_sc: int):
    assert k == 8, "kernel is specialized to K == 8 mixture components"
    per_w = n_sc // _NW
    assert per_w * _NW == n_sc
    chunk = min(_CHUNK, per_w)
    n_ch = per_w // chunk
    assert n_ch * chunk == per_w
    assert chunk % _LANES == 0 and n % _LANES == 0
    tile = k * _LANES

    mesh = plsc.VectorSubcoreMesh(
        core_axis_name="c", subcore_axis_name="s", num_cores=_NC, num_subcores=_NS
    )

    @functools.partial(
        pl.kernel,
        mesh=mesh,
        compiler_params=pltpu.CompilerParams(needs_layout_passes=False),
        out_type=jax.ShapeDtypeStruct((n_sc,), jnp.float32),
        scratch_types=[
            pltpu.VMEM((chunk * k,), jnp.float32),
            pltpu.VMEM((chunk * k,), jnp.float32),
            pltpu.VMEM((chunk,), jnp.int32),
            pltpu.VMEM((chunk,), jnp.int32),
            pltpu.VMEM((chunk,), jnp.float32),
            pltpu.VMEM((chunk,), jnp.float32),
            pltpu.VMEM((2 * k,), jnp.float32),
            pltpu.SemaphoreType.DMA,
            pltpu.SemaphoreType.DMA,
            pltpu.SemaphoreType.DMA,
            pltpu.SemaphoreType.DMA,
        ],
    )
    def sc_kernel(eps_hbm, ms_hbm, tab_hbm, out_hbm,
                  eb0, eb1, mb0, mb1, ob0, ob1, tab_v, ls0, ls1, ss0, ss1):
        eb = (eb0, eb1)
        mb = (mb0, mb1)
        ob = (ob0, ob1)
        lsem = (ls0, ls1)
        ssem = (ss0, ss1)

        wid = lax.axis_index("s") * _NC + lax.axis_index("c")
        base = wid * per_w

        pltpu.sync_copy(tab_hbm, tab_v)
        tab = tab_v[...]

        iota = lax.iota(jnp.int32, _L)

        def start_loads(c, b):
            off = base + c * chunk
            d1 = pltpu.async_copy(
                eps_hbm.at[pl.ds(off * k, chunk * k)], eb[b], lsem[b]
            )
            d2 = pltpu.async_copy(ms_hbm.at[pl.ds(off, chunk)], mb[b], lsem[b])
            return (d1, d2)

        def compute(b):
            @plsc.parallel_loop(0, chunk, _L, unroll=8)
            def p(j):
                sl = pl.ds(j, _L)
                m = mb[b][sl]
                lo = _take(tab, m)
                sc = _take(tab, m + k)
                s = (j // _LANES) * tile + (j % _LANES)
                lidx = lax.shift_left(m, 7) + (s + iota)
                g = plsc.load_gather(eb[b], [lidx])
                ob[b][sl] = lo + sc * g

        def start_store(c, b):
            off = base + c * chunk
            return pltpu.async_copy(ob[b], out_hbm.at[pl.ds(off, chunk)], ssem[b])

        load_d = [None, None]
        store_d = [None, None]
        load_d[0] = start_loads(0, 0)
        if n_ch > 1:
            load_d[1] = start_loads(1, 1)
        for c in range(n_ch):
            b = c & 1
            for d in load_d[b]:
                d.wait()
            if store_d[b] is not None:
                store_d[b].wait()
                store_d[b] = None
            compute(b)
            store_d[b] = start_store(c, b)
            if c + 2 < n_ch:
                load_d[b] = start_loads(c + 2, b)
        for b in range(2):
            if store_d[b] is not None:
                store_d[b].wait()

    return sc_kernel


@functools.lru_cache(maxsize=None)
def _build_tc_kernel(n: int, k: int, n_sc: int):
    t_all = n // _LANES
    t_sc = n_sc // _LANES
    t_tc = t_all - t_sc
    rb = min(_TC_BLOCK, t_tc)
    assert t_tc % rb == 0 and t_sc % rb == 0
    grid = (t_tc // rb,)
    off_b = t_sc // rb

    def tc_body(eps_ref, ms_ref, loc_ref, scale_ref, out_ref):
        m = ms_ref[...]
        acc = loc_ref[0] + scale_ref[0] * eps_ref[:, 0, :]
        for mm in range(1, k):
            acc = jnp.where(
                m == mm, loc_ref[mm] + scale_ref[mm] * eps_ref[:, mm, :], acc
            )
        out_ref[...] = acc

    return pl.pallas_call(
        tc_body,
        grid=grid,
        in_specs=[
            pl.BlockSpec((rb, k, _LANES), lambda b: (off_b + b, 0, 0)),
            pl.BlockSpec((rb, _LANES), lambda b: (off_b + b, 0)),
            pl.BlockSpec(memory_space=pltpu.SMEM),
            pl.BlockSpec(memory_space=pltpu.SMEM),
        ],
        out_specs=pl.BlockSpec((rb, _LANES), lambda b: (b, 0)),
        out_shape=jax.ShapeDtypeStruct((t_tc, _LANES), jnp.float32),
    )


def kernel(eps, ms, location, scale):
    n, k = eps.shape
    n_sc = (n // 32) * _SC_UNITS
    assert n_sc % (_NW * _LANES) == 0
    eps3 = eps.reshape(n // _LANES, _LANES, k).transpose(0, 2, 1)
    ms_i = ms.astype(jnp.int32)
    loc_f = location.astype(jnp.float32)
    scale_f = scale.astype(jnp.float32)
    tab = jnp.concatenate([loc_f, scale_f])

    sc_out = _build_sc_kernel(n, k, n_sc)(
        eps3.reshape(n * k), ms_i, tab
    )
    tc_out = _build_tc_kernel(n, k, n_sc)(
        eps3, ms_i.reshape(n // _LANES, _LANES), loc_f, scale_f
    )
    return jnp.concatenate([sc_out, tc_out.reshape(n - n_sc)])
